# baseline (speedup 1.0000x reference)
import jax
import jax.numpy as jnp
from jax import lax
from jax.experimental import pallas as pl
from jax.experimental.pallas import tpu as pltpu
from jax.experimental.pallas import tpu_sc as plsc

_NC = 2
_NS = 16
_L = 16

_CH = 256
_NCHUNK = 8
_PW = _CH * _NCHUNK


def _bcast_lane(vec, p):
    sel = jnp.full((_L, 1), p, jnp.int32)
    return lax.gather(
        vec,
        sel,
        lax.GatherDimensionNumbers(
            offset_dims=(),
            collapsed_slice_dims=(0,),
            start_index_map=(0,),
        ),
        slice_sizes=(1,),
        mode=lax.GatherScatterMode.PROMISE_IN_BOUNDS,
    )


def _tent_body(dflat_hbm, samp_hbm, out_hbm, in_v, samp_v, buf0, buf1, sems):
    wid = lax.axis_index("s") * _NC + lax.axis_index("c")
    nd = wid // 2
    half = wid % 2
    base = half * _PW

    pltpu.sync_copy(dflat_hbm.at[pl.ds(wid * (2 * _PW), 2 * _PW)], in_v)
    pltpu.sync_copy(samp_hbm, samp_v)
    s_vregs = [samp_v[pl.ds(_L * k, _L)] for k in range(4)]
    bufs = (buf0, buf1)

    def chunk_compute(c, buf):
        @pl.loop(0, 2 * _CH // _L)
        def _octet(q):
            raw = in_v[pl.ds(c * (2 * _CH) + q * _L, _L)]
            for p in range(_L // 2):
                xb = _bcast_lane(raw, 2 * p)
                yb = _bcast_lane(raw, 2 * p + 1)
                r = q * (_L // 2) + p
                for k in range(4):
                    u = yb - s_vregs[k]
                    v = s_vregs[k] - xb
                    buf[r, pl.ds(_L * k, _L)] = jnp.maximum(
                        jnp.minimum(u, v), 0.0
                    )

    handles = []
    for c in range(_NCHUNK):
        b = c % 2
        if c >= 2:
            handles[c - 2].wait()
        chunk_compute(c, bufs[b])
        handles.append(
            pltpu.async_copy(
                bufs[b],
                out_hbm.at[nd, pl.ds(base + c * _CH, _CH), :],
                sems.at[b],
            )
        )
    handles[-2].wait()
    handles[-1].wait()


def kernel(diagrams, samples):
    n, P, _ = diagrams.shape
    S = samples.shape[0]
    fn = pl.kernel(
        _tent_body,
        out_type=jax.ShapeDtypeStruct((n, P, S), jnp.float32),
        mesh=plsc.VectorSubcoreMesh(core_axis_name="c", subcore_axis_name="s"),
        scratch_types=[
            pltpu.VMEM((2 * _PW,), jnp.float32),
            pltpu.VMEM((S,), jnp.float32),
            pltpu.VMEM((_CH, S), jnp.float32),
            pltpu.VMEM((_CH, S), jnp.float32),
            pltpu.SemaphoreType.DMA((2,)),
        ],
    )
    return fn(diagrams.reshape(-1), samples)
